# baseline (speedup 1.0000x reference)
import functools

import jax
import jax.numpy as jnp
from jax import lax
from jax.experimental import pallas as pl
from jax.experimental.pallas import tpu as pltpu
from jax.experimental.pallas import tpu_sc as plsc

N = 10000
D_IN = 128
D_H = 64
E = 320000

NC = 2
NS = 16
L = 16
NW = NC * NS
CH = 128
NBUF = 4
K = -(-(-(-E // (NW * CH))) // NBUF) * NBUF
EPAD = NW * K * CH
ACC_ROWS = 10240
RPT_Z = ACC_ROWS // NS
RPT_O = 624
TAIL_O = N - NS * RPT_O

BR = 512
BM = 512
BN = 8192


def _sc_mesh():
    return plsc.VectorSubcoreMesh(core_axis_name="c", subcore_axis_name="s")


_SC_PARAMS = pltpu.CompilerParams(use_tc_tiling_on_sc=False)


def _deg_counts(dstp):
    @functools.partial(
        pl.kernel,
        out_type=jax.ShapeDtypeStruct((NC, N, L), jnp.float32),
        mesh=_sc_mesh(),
        scratch_types=[
            pltpu.VMEM((K, CH), jnp.int32),
            pltpu.VMEM((CH, L), jnp.float32),
            pltpu.VMEM((L, L), jnp.float32),
            pltpu.VMEM_SHARED((ACC_ROWS, L), jnp.float32),
        ],
        compiler_params=_SC_PARAMS,
    )
    def k(dst_hbm, out_hbm, didx, ones_v, zbuf, acc):
        c = lax.axis_index("c")
        s = lax.axis_index("s")
        wid = c * NS + s
        pltpu.sync_copy(dst_hbm.at[wid], didx)

        @pl.loop(0, CH)
        def _(r):
            ones_v[r, :] = jnp.full((L,), 1.0, jnp.float32)

        @pl.loop(0, L)
        def _(r):
            zbuf[r, :] = jnp.zeros((L,), jnp.float32)

        @pl.loop(0, RPT_Z // L)
        def _(t):
            pltpu.sync_copy(zbuf, acc.at[pl.ds(s * RPT_Z + t * L, L)])

        plsc.subcore_barrier()

        @pl.loop(0, K)
        def _(j):
            pltpu.sync_copy(ones_v, acc.at[didx.at[j]], add=True)

        plsc.subcore_barrier()
        pltpu.sync_copy(acc.at[pl.ds(s * RPT_O, RPT_O)],
                        out_hbm.at[c].at[pl.ds(s * RPT_O, RPT_O)])

        @pl.when(s == NS - 1)
        def _():
            pltpu.sync_copy(acc.at[pl.ds(NS * RPT_O, TAIL_O)],
                            out_hbm.at[c].at[pl.ds(NS * RPT_O, TAIL_O)])

    return k(dstp)


def _scatter_partials(u, srcp, dstp, d):
    d2 = d // 2

    @functools.partial(
        pl.kernel,
        out_type=jax.ShapeDtypeStruct((2, NC, N, d2), jnp.float32),
        mesh=_sc_mesh(),
        scratch_types=[
            pltpu.VMEM((K, CH), jnp.int32),
            pltpu.VMEM((K, CH), jnp.int32),
            pltpu.VMEM((CH, d2), jnp.float32),
            pltpu.VMEM((CH, d2), jnp.float32),
            pltpu.VMEM((CH, d2), jnp.float32),
            pltpu.VMEM((CH, d2), jnp.float32),
            pltpu.VMEM((L, d2), jnp.float32),
            pltpu.VMEM_SHARED((N, d2), jnp.float32),
            pltpu.VMEM_SHARED((ACC_ROWS, d2), jnp.float32),
            pltpu.SemaphoreType.DMA,
            pltpu.SemaphoreType.DMA,
            pltpu.SemaphoreType.DMA,
            pltpu.SemaphoreType.DMA,
        ],
        compiler_params=_SC_PARAMS,
    )
    def k(u_hbm, src_hbm, dst_hbm, out_hbm, sidx, didx,
          r0, r1, r2, r3, zbuf, u_s, acc, s0, s1, s2, s3):
        c = lax.axis_index("c")
        s = lax.axis_index("s")
        wid = c * NS + s
        bufs = (r0, r1, r2, r3)
        sems = (s0, s1, s2, s3)
        pltpu.sync_copy(src_hbm.at[wid], sidx)
        pltpu.sync_copy(dst_hbm.at[wid], didx)

        @pl.loop(0, L)
        def _(r):
            @pl.loop(0, d2, step=L)
            def _(q):
                zbuf[r, pl.ds(q, L)] = jnp.zeros((L,), jnp.float32)

        for h in range(2):
            @pl.loop(0, RPT_Z // L)
            def _(t):
                pltpu.sync_copy(zbuf, acc.at[pl.ds(s * RPT_Z + t * L, L)])

            @pl.when(s < NS - 1)
            def _():
                pltpu.sync_copy(
                    u_hbm.at[pl.ds(s * RPT_Z, RPT_Z), pl.ds(h * d2, d2)],
                    u_s.at[pl.ds(s * RPT_Z, RPT_Z)])

            @pl.when(s == NS - 1)
            def _():
                pltpu.sync_copy(
                    u_hbm.at[pl.ds((NS - 1) * RPT_Z, N - (NS - 1) * RPT_Z),
                             pl.ds(h * d2, d2)],
                    u_s.at[pl.ds((NS - 1) * RPT_Z, N - (NS - 1) * RPT_Z)])

            plsc.subcore_barrier()

            for b in range(NBUF):
                pltpu.async_copy(u_s.at[sidx.at[b]], bufs[b], sems[b])

            @pl.loop(0, K, step=NBUF)
            def _(j):
                for b in range(NBUF):
                    pltpu.make_async_copy(
                        u_hbm.at[pl.ds(0, CH), pl.ds(0, d2)], bufs[b],
                        sems[b]).wait()
                    pltpu.sync_copy(bufs[b], acc.at[didx.at[j + b]], add=True)

                    @pl.when(j + b + NBUF < K)
                    def _():
                        pltpu.async_copy(u_s.at[sidx.at[j + b + NBUF]],
                                         bufs[b], sems[b])

            plsc.subcore_barrier()
            pltpu.sync_copy(acc.at[pl.ds(s * RPT_O, RPT_O)],
                            out_hbm.at[h].at[c].at[pl.ds(s * RPT_O, RPT_O)])

            @pl.when(s == NS - 1)
            def _():
                pltpu.sync_copy(acc.at[pl.ds(NS * RPT_O, TAIL_O)],
                                out_hbm.at[h].at[c].at[pl.ds(NS * RPT_O, TAIL_O)])

            plsc.subcore_barrier()

    return k(u, srcp, dstp)


def _dis_block(cnt):
    deg = 1.0 + cnt[0] + cnt[1]
    return lax.rsqrt(deg)[:, 0:1]


def _encode0(x, w0, counts):
    def body(x_ref, w_ref, cnt_ref, o_ref):
        dis = _dis_block(cnt_ref[...])
        h = jnp.dot(x_ref[...], w_ref[...], preferred_element_type=jnp.float32,
                    precision=lax.Precision.HIGHEST)
        o_ref[...] = h * dis

    return pl.pallas_call(
        body,
        grid=(pl.cdiv(N, BR),),
        in_specs=[
            pl.BlockSpec((BR, D_IN), lambda i: (i, 0)),
            pl.BlockSpec((D_IN, D_H), lambda i: (0, 0)),
            pl.BlockSpec((NC, BR, L), lambda i: (0, i, 0)),
        ],
        out_specs=pl.BlockSpec((BR, D_H), lambda i: (i, 0)),
        out_shape=jax.ShapeDtypeStruct((N, D_H), jnp.float32),
    )(x, w0, counts)


def _mid(p, u, counts, b, w, dn):
    def body(p_ref, u_ref, cnt_ref, b_ref, *rest):
        if w is None:
            z_ref, un_ref = rest
        else:
            w_ref, z_ref, un_ref = rest
        dis = _dis_block(cnt_ref[...])
        psum = jnp.concatenate(
            [p_ref[0, 0] + p_ref[0, 1], p_ref[1, 0] + p_ref[1, 1]], axis=-1)
        zb = jnp.maximum((psum + u_ref[...]) * dis + b_ref[...], 0.0)
        z_ref[...] = zb
        if w is None:
            un_ref[...] = zb * dis
        else:
            un_ref[...] = jnp.dot(zb, w_ref[...], preferred_element_type=jnp.float32,
                                  precision=lax.Precision.HIGHEST) * dis

    in_specs = [
        pl.BlockSpec((2, NC, BR, D_H // 2), lambda i: (0, 0, i, 0)),
        pl.BlockSpec((BR, D_H), lambda i: (i, 0)),
        pl.BlockSpec((NC, BR, L), lambda i: (0, i, 0)),
        pl.BlockSpec((1, D_H), lambda i: (0, 0)),
    ]
    operands = [p, u, counts, b]
    if w is not None:
        in_specs.append(pl.BlockSpec((D_H, dn), lambda i: (0, 0)))
        operands.append(w)
    return pl.pallas_call(
        body,
        grid=(pl.cdiv(N, BR),),
        in_specs=in_specs,
        out_specs=[
            pl.BlockSpec((BR, D_H), lambda i: (i, 0)),
            pl.BlockSpec((BR, dn), lambda i: (i, 0)),
        ],
        out_shape=[
            jax.ShapeDtypeStruct((N, D_H), jnp.float32),
            jax.ShapeDtypeStruct((N, dn), jnp.float32),
        ],
    )(*operands)


def _attr_out(q, v2, counts, wd, bd):
    def body(q_ref, v_ref, cnt_ref, w_ref, b_ref, o_ref):
        dis = _dis_block(cnt_ref[...])
        qsum = jnp.concatenate(
            [q_ref[0, 0] + q_ref[0, 1], q_ref[1, 0] + q_ref[1, 1]], axis=-1)
        t = (qsum + v_ref[...]) * dis
        o_ref[...] = jnp.maximum(
            jnp.dot(t, w_ref[...], preferred_element_type=jnp.float32,
                    precision=lax.Precision.HIGHEST) + b_ref[...], 0.0)

    return pl.pallas_call(
        body,
        grid=(pl.cdiv(N, BR),),
        in_specs=[
            pl.BlockSpec((2, NC, BR, D_H // 2), lambda i: (0, 0, i, 0)),
            pl.BlockSpec((BR, D_H), lambda i: (i, 0)),
            pl.BlockSpec((NC, BR, L), lambda i: (0, i, 0)),
            pl.BlockSpec((D_H, D_IN), lambda i: (0, 0)),
            pl.BlockSpec((1, D_IN), lambda i: (0, 0)),
        ],
        out_specs=pl.BlockSpec((BR, D_IN), lambda i: (i, 0)),
        out_shape=jax.ShapeDtypeStruct((N, D_IN), jnp.float32),
    )(q, v2, counts, wd, bd)


def _struct(z):
    def body(a_ref, b_ref, o_ref):
        o_ref[...] = lax.dot_general(
            a_ref[...], b_ref[...], (((1,), (1,)), ((), ())),
            preferred_element_type=jnp.float32,
            precision=lax.Precision.HIGHEST)

    return pl.pallas_call(
        body,
        grid=(pl.cdiv(N, BM), pl.cdiv(N, BN)),
        in_specs=[
            pl.BlockSpec((BM, D_H), lambda i, j: (i, 0)),
            pl.BlockSpec((BN, D_H), lambda i, j: (j, 0)),
        ],
        out_specs=pl.BlockSpec((BM, BN), lambda i, j: (i, j)),
        out_shape=jax.ShapeDtypeStruct((N, N), jnp.float32),
    )(z, z)


def kernel(x, edge_index, W0, b0, W1, b1, Wd, bd):
    src, dst = edge_index[0], edge_index[1]
    pad = EPAD - E
    srcp = jnp.concatenate([src, jnp.zeros((pad,), src.dtype)]).reshape(NW, K, CH)
    dstp = jnp.concatenate([dst, jnp.full((pad,), N, dst.dtype)]).reshape(NW, K, CH)

    counts = _deg_counts(dstp)
    u0 = _encode0(x, W0, counts)
    p = _scatter_partials(u0, srcp, dstp, D_H)
    _, u1 = _mid(p, u0, counts, b0.reshape(1, D_H), W1, D_H)
    q = _scatter_partials(u1, srcp, dstp, D_H)
    z, v2 = _mid(q, u1, counts, b1.reshape(1, D_H), None, D_H)
    r = _scatter_partials(v2, srcp, dstp, D_H)
    attr_recon = _attr_out(r, v2, counts, Wd, bd.reshape(1, D_IN))
    struct_recon = _struct(z)
    return (struct_recon, attr_recon)


# speedup vs baseline: 1.2973x; 1.2973x over previous
import functools

import jax
import jax.numpy as jnp
from jax import lax
from jax.experimental import pallas as pl
from jax.experimental.pallas import tpu as pltpu
from jax.experimental.pallas import tpu_sc as plsc

N = 10000
D_IN = 128
D_H = 64
E = 320000

NC = 2
NS = 16
L = 16
NW = NC * NS
CH = 128
NBUF = 4
K = -(-(-(-E // (NW * CH))) // NBUF) * NBUF
EPAD = NW * K * CH
ACC_ROWS = 10240
RPT_Z = ACC_ROWS // NS
RPT_O = 624
TAIL_O = N - NS * RPT_O

BR = 512
BM = BN = 2048


def _sc_mesh():
    return plsc.VectorSubcoreMesh(core_axis_name="c", subcore_axis_name="s")


_SC_PARAMS = pltpu.CompilerParams(use_tc_tiling_on_sc=False)


def _deg_counts(dstp):
    @functools.partial(
        pl.kernel,
        out_type=jax.ShapeDtypeStruct((NC, N, L), jnp.float32),
        mesh=_sc_mesh(),
        scratch_types=[
            pltpu.VMEM((K, CH), jnp.int32),
            pltpu.VMEM((CH, L), jnp.float32),
            pltpu.VMEM((L, L), jnp.float32),
            pltpu.VMEM_SHARED((ACC_ROWS, L), jnp.float32),
        ],
        compiler_params=_SC_PARAMS,
    )
    def k(dst_hbm, out_hbm, didx, ones_v, zbuf, acc):
        c = lax.axis_index("c")
        s = lax.axis_index("s")
        wid = c * NS + s
        pltpu.sync_copy(dst_hbm.at[wid], didx)

        @pl.loop(0, CH)
        def _(r):
            ones_v[r, :] = jnp.full((L,), 1.0, jnp.float32)

        @pl.loop(0, L)
        def _(r):
            zbuf[r, :] = jnp.zeros((L,), jnp.float32)

        @pl.loop(0, RPT_Z // L)
        def _(t):
            pltpu.sync_copy(zbuf, acc.at[pl.ds(s * RPT_Z + t * L, L)])

        plsc.subcore_barrier()

        @pl.loop(0, K)
        def _(j):
            pltpu.sync_copy(ones_v, acc.at[didx.at[j]], add=True)

        plsc.subcore_barrier()
        pltpu.sync_copy(acc.at[pl.ds(s * RPT_O, RPT_O)],
                        out_hbm.at[c].at[pl.ds(s * RPT_O, RPT_O)])

        @pl.when(s == NS - 1)
        def _():
            pltpu.sync_copy(acc.at[pl.ds(NS * RPT_O, TAIL_O)],
                            out_hbm.at[c].at[pl.ds(NS * RPT_O, TAIL_O)])

    return k(dstp)


def _scatter_partials(u, srcp, dstp, d):
    d2 = d // 2

    @functools.partial(
        pl.kernel,
        out_type=jax.ShapeDtypeStruct((NC, N, d), jnp.float32),
        mesh=_sc_mesh(),
        scratch_types=[
            pltpu.VMEM((K, CH), jnp.int32),
            pltpu.VMEM((K, CH), jnp.int32),
            pltpu.VMEM((CH, d2), jnp.float32),
            pltpu.VMEM((CH, d2), jnp.float32),
            pltpu.VMEM((CH, d2), jnp.float32),
            pltpu.VMEM((CH, d2), jnp.float32),
            pltpu.VMEM((L, d2), jnp.float32),
            pltpu.VMEM_SHARED((N, d2), jnp.float32),
            pltpu.VMEM_SHARED((ACC_ROWS, d2), jnp.float32),
            pltpu.SemaphoreType.DMA,
            pltpu.SemaphoreType.DMA,
            pltpu.SemaphoreType.DMA,
            pltpu.SemaphoreType.DMA,
        ],
        compiler_params=_SC_PARAMS,
    )
    def k(u_hbm, src_hbm, dst_hbm, out_hbm, sidx, didx,
          r0, r1, r2, r3, zbuf, u_s, acc, s0, s1, s2, s3):
        c = lax.axis_index("c")
        s = lax.axis_index("s")
        wid = c * NS + s
        bufs = (r0, r1, r2, r3)
        sems = (s0, s1, s2, s3)
        pltpu.sync_copy(src_hbm.at[wid], sidx)
        pltpu.sync_copy(dst_hbm.at[wid], didx)

        @pl.loop(0, L)
        def _(r):
            @pl.loop(0, d2, step=L)
            def _(q):
                zbuf[r, pl.ds(q, L)] = jnp.zeros((L,), jnp.float32)

        for h in range(2):
            @pl.loop(0, RPT_Z // L)
            def _(t):
                pltpu.sync_copy(zbuf, acc.at[pl.ds(s * RPT_Z + t * L, L)])

            @pl.when(s < NS - 1)
            def _():
                pltpu.sync_copy(
                    u_hbm.at[pl.ds(s * RPT_Z, RPT_Z), pl.ds(h * d2, d2)],
                    u_s.at[pl.ds(s * RPT_Z, RPT_Z)])

            @pl.when(s == NS - 1)
            def _():
                pltpu.sync_copy(
                    u_hbm.at[pl.ds((NS - 1) * RPT_Z, N - (NS - 1) * RPT_Z),
                             pl.ds(h * d2, d2)],
                    u_s.at[pl.ds((NS - 1) * RPT_Z, N - (NS - 1) * RPT_Z)])

            plsc.subcore_barrier()

            for b in range(NBUF):
                pltpu.async_copy(u_s.at[sidx.at[b]], bufs[b], sems[b])

            @pl.loop(0, K, step=NBUF)
            def _(j):
                for b in range(NBUF):
                    pltpu.make_async_copy(
                        u_hbm.at[pl.ds(0, CH), pl.ds(0, d2)], bufs[b],
                        sems[b]).wait()
                    pltpu.sync_copy(bufs[b], acc.at[didx.at[j + b]], add=True)

                    @pl.when(j + b + NBUF < K)
                    def _():
                        pltpu.async_copy(u_s.at[sidx.at[j + b + NBUF]],
                                         bufs[b], sems[b])

            plsc.subcore_barrier()
            pltpu.sync_copy(
                acc.at[pl.ds(s * RPT_O, RPT_O)],
                out_hbm.at[c].at[pl.ds(s * RPT_O, RPT_O), pl.ds(h * d2, d2)])

            @pl.when(s == NS - 1)
            def _():
                pltpu.sync_copy(
                    acc.at[pl.ds(NS * RPT_O, TAIL_O)],
                    out_hbm.at[c].at[pl.ds(NS * RPT_O, TAIL_O),
                                     pl.ds(h * d2, d2)])

            plsc.subcore_barrier()

    return k(u, srcp, dstp)


def _dis_block(cnt):
    deg = 1.0 + cnt[0] + cnt[1]
    return lax.rsqrt(deg)[:, 0:1]


def _encode0(x, w0, counts):
    def body(x_ref, w_ref, cnt_ref, o_ref):
        dis = _dis_block(cnt_ref[...])
        h = jnp.dot(x_ref[...], w_ref[...], preferred_element_type=jnp.float32,
                    precision=lax.Precision.HIGHEST)
        o_ref[...] = h * dis

    return pl.pallas_call(
        body,
        grid=(pl.cdiv(N, BR),),
        in_specs=[
            pl.BlockSpec((BR, D_IN), lambda i: (i, 0)),
            pl.BlockSpec((D_IN, D_H), lambda i: (0, 0)),
            pl.BlockSpec((NC, BR, L), lambda i: (0, i, 0)),
        ],
        out_specs=pl.BlockSpec((BR, D_H), lambda i: (i, 0)),
        out_shape=jax.ShapeDtypeStruct((N, D_H), jnp.float32),
    )(x, w0, counts)


def _mid(p, u, counts, b, w, dn):
    def body(p_ref, u_ref, cnt_ref, b_ref, *rest):
        if w is None:
            z_ref, un_ref = rest
        else:
            w_ref, z_ref, un_ref = rest
        dis = _dis_block(cnt_ref[...])
        zb = jnp.maximum((p_ref[0] + p_ref[1] + u_ref[...]) * dis + b_ref[...], 0.0)
        z_ref[...] = zb
        if w is None:
            un_ref[...] = zb * dis
        else:
            un_ref[...] = jnp.dot(zb, w_ref[...], preferred_element_type=jnp.float32,
                                  precision=lax.Precision.HIGHEST) * dis

    in_specs = [
        pl.BlockSpec((NC, BR, D_H), lambda i: (0, i, 0)),
        pl.BlockSpec((BR, D_H), lambda i: (i, 0)),
        pl.BlockSpec((NC, BR, L), lambda i: (0, i, 0)),
        pl.BlockSpec((1, D_H), lambda i: (0, 0)),
    ]
    operands = [p, u, counts, b]
    if w is not None:
        in_specs.append(pl.BlockSpec((D_H, dn), lambda i: (0, 0)))
        operands.append(w)
    return pl.pallas_call(
        body,
        grid=(pl.cdiv(N, BR),),
        in_specs=in_specs,
        out_specs=[
            pl.BlockSpec((BR, D_H), lambda i: (i, 0)),
            pl.BlockSpec((BR, dn), lambda i: (i, 0)),
        ],
        out_shape=[
            jax.ShapeDtypeStruct((N, D_H), jnp.float32),
            jax.ShapeDtypeStruct((N, dn), jnp.float32),
        ],
    )(*operands)


def _attr_out(q, v2, counts, wd, bd):
    def body(q_ref, v_ref, cnt_ref, w_ref, b_ref, o_ref):
        dis = _dis_block(cnt_ref[...])
        t = (q_ref[0] + q_ref[1] + v_ref[...]) * dis
        o_ref[...] = jnp.maximum(
            jnp.dot(t, w_ref[...], preferred_element_type=jnp.float32,
                    precision=lax.Precision.HIGHEST) + b_ref[...], 0.0)

    return pl.pallas_call(
        body,
        grid=(pl.cdiv(N, BR),),
        in_specs=[
            pl.BlockSpec((NC, BR, D_H), lambda i: (0, i, 0)),
            pl.BlockSpec((BR, D_H), lambda i: (i, 0)),
            pl.BlockSpec((NC, BR, L), lambda i: (0, i, 0)),
            pl.BlockSpec((D_H, D_IN), lambda i: (0, 0)),
            pl.BlockSpec((1, D_IN), lambda i: (0, 0)),
        ],
        out_specs=pl.BlockSpec((BR, D_IN), lambda i: (i, 0)),
        out_shape=jax.ShapeDtypeStruct((N, D_IN), jnp.float32),
    )(q, v2, counts, wd, bd)


def _struct(z):
    def body(a_ref, b_ref, o_ref):
        o_ref[...] = lax.dot_general(
            a_ref[...], b_ref[...], (((1,), (1,)), ((), ())),
            preferred_element_type=jnp.float32,
            precision=lax.Precision.HIGHEST)

    return pl.pallas_call(
        body,
        grid=(pl.cdiv(N, BM), pl.cdiv(N, BN)),
        in_specs=[
            pl.BlockSpec((BM, D_H), lambda i, j: (i, 0)),
            pl.BlockSpec((BN, D_H), lambda i, j: (j, 0)),
        ],
        out_specs=pl.BlockSpec((BM, BN), lambda i, j: (i, j)),
        out_shape=jax.ShapeDtypeStruct((N, N), jnp.float32),
    )(z, z)


def kernel(x, edge_index, W0, b0, W1, b1, Wd, bd):
    src, dst = edge_index[0], edge_index[1]
    pad = EPAD - E
    srcp = jnp.concatenate([src, jnp.zeros((pad,), src.dtype)]).reshape(NW, K, CH)
    dstp = jnp.concatenate([dst, jnp.full((pad,), N, dst.dtype)]).reshape(NW, K, CH)

    counts = _deg_counts(dstp)
    u0 = _encode0(x, W0, counts)
    p = _scatter_partials(u0, srcp, dstp, D_H)
    _, u1 = _mid(p, u0, counts, b0.reshape(1, D_H), W1, D_H)
    q = _scatter_partials(u1, srcp, dstp, D_H)
    z, v2 = _mid(q, u1, counts, b1.reshape(1, D_H), None, D_H)
    r = _scatter_partials(v2, srcp, dstp, D_H)
    attr_recon = _attr_out(r, v2, counts, Wd, bd.reshape(1, D_IN))
    struct_recon = _struct(z)
    return (struct_recon, attr_recon)


# speedup vs baseline: 1.3096x; 1.0095x over previous
import functools

import jax
import jax.numpy as jnp
from jax import lax
from jax.experimental import pallas as pl
from jax.experimental.pallas import tpu as pltpu
from jax.experimental.pallas import tpu_sc as plsc

N = 10000
D_IN = 128
D_H = 64
E = 320000

NC = 2
NS = 16
L = 16
NW = NC * NS
CH = 128
NBUF = 4
K = -(-(-(-E // (NW * CH))) // NBUF) * NBUF
EPAD = NW * K * CH
ACC_ROWS = 10240
RPT_Z = ACC_ROWS // NS
RPT_O = 624
TAIL_O = N - NS * RPT_O

BR = 512
BM = BN = 2048


def _sc_mesh():
    return plsc.VectorSubcoreMesh(core_axis_name="c", subcore_axis_name="s")


_SC_PARAMS = pltpu.CompilerParams(use_tc_tiling_on_sc=False)


def _deg_counts(dstp):
    @functools.partial(
        pl.kernel,
        out_type=jax.ShapeDtypeStruct((NC, N, L), jnp.float32),
        mesh=_sc_mesh(),
        scratch_types=[
            pltpu.VMEM((K, CH), jnp.int32),
            pltpu.VMEM((CH, L), jnp.float32),
            pltpu.VMEM((L, L), jnp.float32),
            pltpu.VMEM_SHARED((ACC_ROWS, L), jnp.float32),
        ],
        compiler_params=_SC_PARAMS,
    )
    def k(dst_hbm, out_hbm, didx, ones_v, zbuf, acc):
        c = lax.axis_index("c")
        s = lax.axis_index("s")
        wid = c * NS + s
        pltpu.sync_copy(dst_hbm.at[wid], didx)

        @pl.loop(0, CH)
        def _(r):
            ones_v[r, :] = jnp.full((L,), 1.0, jnp.float32)

        @pl.loop(0, L)
        def _(r):
            zbuf[r, :] = jnp.zeros((L,), jnp.float32)

        @pl.loop(0, RPT_Z // L)
        def _(t):
            pltpu.sync_copy(zbuf, acc.at[pl.ds(s * RPT_Z + t * L, L)])

        plsc.subcore_barrier()

        @pl.loop(0, K)
        def _(j):
            pltpu.sync_copy(ones_v, acc.at[didx.at[j]], add=True)

        plsc.subcore_barrier()
        pltpu.sync_copy(acc.at[pl.ds(s * RPT_O, RPT_O)],
                        out_hbm.at[c].at[pl.ds(s * RPT_O, RPT_O)])

        @pl.when(s == NS - 1)
        def _():
            pltpu.sync_copy(acc.at[pl.ds(NS * RPT_O, TAIL_O)],
                            out_hbm.at[c].at[pl.ds(NS * RPT_O, TAIL_O)])

    return k(dstp)


def _scatter_partials(u, srcp, dstp, d):
    d2 = d // 2

    @functools.partial(
        pl.kernel,
        out_type=jax.ShapeDtypeStruct((NC, N, d), jnp.float32),
        mesh=_sc_mesh(),
        scratch_types=[
            pltpu.VMEM((K, CH), jnp.int32),
            pltpu.VMEM((K, CH), jnp.int32),
            pltpu.VMEM((CH, d2), jnp.float32),
            pltpu.VMEM((CH, d2), jnp.float32),
            pltpu.VMEM((CH, d2), jnp.float32),
            pltpu.VMEM((CH, d2), jnp.float32),
            pltpu.VMEM((L, d2), jnp.float32),
            pltpu.VMEM_SHARED((N, d2), jnp.float32),
            pltpu.VMEM_SHARED((ACC_ROWS, d2), jnp.float32),
            pltpu.SemaphoreType.DMA,
            pltpu.SemaphoreType.DMA,
            pltpu.SemaphoreType.DMA,
            pltpu.SemaphoreType.DMA,
        ],
        compiler_params=_SC_PARAMS,
    )
    def k(u_hbm, src_hbm, dst_hbm, out_hbm, sidx, didx,
          r0, r1, r2, r3, zbuf, u_s, acc, s0, s1, s2, s3):
        c = lax.axis_index("c")
        s = lax.axis_index("s")
        wid = c * NS + s
        bufs = (r0, r1, r2, r3)
        sems = (s0, s1, s2, s3)
        pltpu.sync_copy(src_hbm.at[wid], sidx)
        pltpu.sync_copy(dst_hbm.at[wid], didx)

        @pl.loop(0, L)
        def _(r):
            @pl.loop(0, d2, step=L)
            def _(q):
                zbuf[r, pl.ds(q, L)] = jnp.zeros((L,), jnp.float32)

        for h in range(2):
            @pl.loop(0, RPT_Z // L)
            def _(t):
                pltpu.sync_copy(zbuf, acc.at[pl.ds(s * RPT_Z + t * L, L)])

            @pl.when(s < NS - 1)
            def _():
                pltpu.sync_copy(
                    u_hbm.at[pl.ds(s * RPT_Z, RPT_Z), pl.ds(h * d2, d2)],
                    u_s.at[pl.ds(s * RPT_Z, RPT_Z)])

            @pl.when(s == NS - 1)
            def _():
                pltpu.sync_copy(
                    u_hbm.at[pl.ds((NS - 1) * RPT_Z, N - (NS - 1) * RPT_Z),
                             pl.ds(h * d2, d2)],
                    u_s.at[pl.ds((NS - 1) * RPT_Z, N - (NS - 1) * RPT_Z)])

            plsc.subcore_barrier()

            for b in range(NBUF):
                pltpu.async_copy(u_s.at[sidx.at[b]], bufs[b], sems[b])

            @pl.loop(0, K, step=NBUF)
            def _(j):
                for b in range(NBUF):
                    pltpu.make_async_copy(
                        u_hbm.at[pl.ds(0, CH), pl.ds(0, d2)], bufs[b],
                        sems[b]).wait()
                    pltpu.sync_copy(bufs[b], acc.at[didx.at[j + b]], add=True)

                    @pl.when(j + b + NBUF < K)
                    def _():
                        pltpu.async_copy(u_s.at[sidx.at[j + b + NBUF]],
                                         bufs[b], sems[b])

            plsc.subcore_barrier()
            pltpu.sync_copy(
                acc.at[pl.ds(s * RPT_O, RPT_O)],
                out_hbm.at[c].at[pl.ds(s * RPT_O, RPT_O), pl.ds(h * d2, d2)])

            @pl.when(s == NS - 1)
            def _():
                pltpu.sync_copy(
                    acc.at[pl.ds(NS * RPT_O, TAIL_O)],
                    out_hbm.at[c].at[pl.ds(NS * RPT_O, TAIL_O),
                                     pl.ds(h * d2, d2)])

            plsc.subcore_barrier()

    return k(u, srcp, dstp)


def _dis_block(cnt):
    deg = 1.0 + cnt[0] + cnt[1]
    return lax.rsqrt(deg)[:, 0:1]


def _encode0(x, w0):
    def body(x_ref, w_ref, o_ref):
        o_ref[...] = jnp.dot(x_ref[...], w_ref[...],
                             preferred_element_type=jnp.float32,
                             precision=lax.Precision.HIGHEST)

    return pl.pallas_call(
        body,
        grid=(pl.cdiv(N, BR),),
        in_specs=[
            pl.BlockSpec((BR, D_IN), lambda i: (i, 0)),
            pl.BlockSpec((D_IN, D_H), lambda i: (0, 0)),
        ],
        out_specs=pl.BlockSpec((BR, D_H), lambda i: (i, 0)),
        out_shape=jax.ShapeDtypeStruct((N, D_H), jnp.float32),
    )(x, w0)


def _disv_scale(counts, hm):
    def body(cnt_ref, h_ref, dv_ref, u0_ref):
        dis = _dis_block(cnt_ref[...])
        dv = jnp.broadcast_to(dis, (BR, D_H))
        dv_ref[...] = dv
        u0_ref[...] = h_ref[...] * dv

    return pl.pallas_call(
        body,
        grid=(pl.cdiv(N, BR),),
        in_specs=[
            pl.BlockSpec((NC, BR, L), lambda i: (0, i, 0)),
            pl.BlockSpec((BR, D_H), lambda i: (i, 0)),
        ],
        out_specs=[
            pl.BlockSpec((BR, D_H), lambda i: (i, 0)),
            pl.BlockSpec((BR, D_H), lambda i: (i, 0)),
        ],
        out_shape=[
            jax.ShapeDtypeStruct((N, D_H), jnp.float32),
            jax.ShapeDtypeStruct((N, D_H), jnp.float32),
        ],
    )(counts, hm)


def _mid(p, u, disv, b, w, dn):
    def body(p_ref, u_ref, dv_ref, b_ref, *rest):
        if w is None:
            z_ref, un_ref = rest
        else:
            w_ref, z_ref, un_ref = rest
        dis = dv_ref[...]
        zb = jnp.maximum((p_ref[0] + p_ref[1] + u_ref[...]) * dis + b_ref[...], 0.0)
        z_ref[...] = zb
        if w is None:
            un_ref[...] = zb * dis
        else:
            un_ref[...] = jnp.dot(zb, w_ref[...], preferred_element_type=jnp.float32,
                                  precision=lax.Precision.HIGHEST) * dis

    in_specs = [
        pl.BlockSpec((NC, BR, D_H), lambda i: (0, i, 0)),
        pl.BlockSpec((BR, D_H), lambda i: (i, 0)),
        pl.BlockSpec((BR, D_H), lambda i: (i, 0)),
        pl.BlockSpec((1, D_H), lambda i: (0, 0)),
    ]
    operands = [p, u, disv, b]
    if w is not None:
        in_specs.append(pl.BlockSpec((D_H, dn), lambda i: (0, 0)))
        operands.append(w)
    return pl.pallas_call(
        body,
        grid=(pl.cdiv(N, BR),),
        in_specs=in_specs,
        out_specs=[
            pl.BlockSpec((BR, D_H), lambda i: (i, 0)),
            pl.BlockSpec((BR, dn), lambda i: (i, 0)),
        ],
        out_shape=[
            jax.ShapeDtypeStruct((N, D_H), jnp.float32),
            jax.ShapeDtypeStruct((N, dn), jnp.float32),
        ],
    )(*operands)


def _attr_out(q, v2, disv, wd, bd):
    def body(q_ref, v_ref, dv_ref, w_ref, b_ref, o_ref):
        t = (q_ref[0] + q_ref[1] + v_ref[...]) * dv_ref[...]
        o_ref[...] = jnp.maximum(
            jnp.dot(t, w_ref[...], preferred_element_type=jnp.float32,
                    precision=lax.Precision.HIGHEST) + b_ref[...], 0.0)

    return pl.pallas_call(
        body,
        grid=(pl.cdiv(N, BR),),
        in_specs=[
            pl.BlockSpec((NC, BR, D_H), lambda i: (0, i, 0)),
            pl.BlockSpec((BR, D_H), lambda i: (i, 0)),
            pl.BlockSpec((BR, D_H), lambda i: (i, 0)),
            pl.BlockSpec((D_H, D_IN), lambda i: (0, 0)),
            pl.BlockSpec((1, D_IN), lambda i: (0, 0)),
        ],
        out_specs=pl.BlockSpec((BR, D_IN), lambda i: (i, 0)),
        out_shape=jax.ShapeDtypeStruct((N, D_IN), jnp.float32),
    )(q, v2, disv, wd, bd)


def _struct(z):
    def body(a_ref, b_ref, o_ref):
        o_ref[...] = lax.dot_general(
            a_ref[...], b_ref[...], (((1,), (1,)), ((), ())),
            preferred_element_type=jnp.float32,
            precision=lax.Precision.HIGHEST)

    return pl.pallas_call(
        body,
        grid=(pl.cdiv(N, BM), pl.cdiv(N, BN)),
        in_specs=[
            pl.BlockSpec((BM, D_H), lambda i, j: (i, 0)),
            pl.BlockSpec((BN, D_H), lambda i, j: (j, 0)),
        ],
        out_specs=pl.BlockSpec((BM, BN), lambda i, j: (i, j)),
        out_shape=jax.ShapeDtypeStruct((N, N), jnp.float32),
    )(z, z)


def kernel(x, edge_index, W0, b0, W1, b1, Wd, bd):
    src, dst = edge_index[0], edge_index[1]
    pad = EPAD - E
    srcp = jnp.concatenate([src, jnp.zeros((pad,), src.dtype)]).reshape(NW, K, CH)
    dstp = jnp.concatenate([dst, jnp.full((pad,), N, dst.dtype)]).reshape(NW, K, CH)

    counts = _deg_counts(dstp)
    hm = _encode0(x, W0)
    disv, u0 = _disv_scale(counts, hm)
    p = _scatter_partials(u0, srcp, dstp, D_H)
    _, u1 = _mid(p, u0, disv, b0.reshape(1, D_H), W1, D_H)
    q = _scatter_partials(u1, srcp, dstp, D_H)
    z, v2 = _mid(q, u1, disv, b1.reshape(1, D_H), None, D_H)
    r = _scatter_partials(v2, srcp, dstp, D_H)
    attr_recon = _attr_out(r, v2, disv, Wd, bd.reshape(1, D_IN))
    struct_recon = _struct(z)
    return (struct_recon, attr_recon)
